# baseline (speedup 1.0000x reference)
import functools

import jax
import jax.numpy as jnp
from jax import lax
from jax.experimental import pallas as pl
from jax.experimental.pallas import tpu as pltpu
from jax.experimental.pallas import tpu_sc as plsc

B, N, C, K = 8, 2048, 128, 16
TM = 256
TN = 512
NUM_IDX = B * N * K


_CHK = 16
_CL = N // _CHK
_R = 8


def _knn_body(xq_ref, xc_ref, pt_ref, idx_ref, lp_ref, dscr):
    b = pl.program_id(0)
    xq = xq_ref[0]
    acc = None
    for c in range(3):
        qc = xq[:, c:c + 1].reshape(TM, 1, 1)
        kc = xc_ref[0, c, :].reshape(1, _CHK, _CL)
        d = qc - kc
        acc = d * d if acc is None else acc + d * d
    dscr[...] = acc
    m0 = jnp.max(acc, axis=2)
    pt = pt_ref[...]
    lp_ref[...] = jnp.where(pt >= 0, pt, 0.01 * pt)
    lane3 = lax.broadcasted_iota(jnp.int32, (TM, _CHK, _CL), 2)
    lane = lax.broadcasted_iota(jnp.int32, (TM, 128), 1)

    def s1(i, car):
        m, cv, cx = car
        w = dscr[...]
        eq = w == m[:, :, None]
        ix = jnp.min(jnp.where(eq, lane3, _CL), axis=2)
        chosen = eq & (lane3 == ix[:, :, None])
        w2 = jnp.where(chosen, -jnp.inf, w)
        dscr[...] = w2
        grp = lane // _CHK
        cv = jnp.where(grp == i, jnp.tile(m, (1, _R)), cv)
        cx = jnp.where(grp == i, jnp.tile(ix, (1, _R)), cx)
        return jnp.max(w2, axis=2), cv, cx

    _, cand_v, cand_ix = lax.fori_loop(
        0, _R, s1,
        (m0, jnp.full((TM, 128), -jnp.inf, jnp.float32),
         jnp.zeros((TM, 128), jnp.int32)))
    cand_g = cand_ix + (lane % _CHK) * _CL

    def s2(j, car):
        w, out = car
        m = jnp.max(w, axis=1, keepdims=True)
        eq = w == m
        sel = jnp.min(jnp.where(eq, cand_g, jnp.int32(1 << 30)),
                      axis=1, keepdims=True)
        keep = (lane == j - 1) & (j > 0)
        out = jnp.where(keep, sel + b * N, out)
        w = jnp.where(eq & (cand_g == sel), -jnp.inf, w)
        return w, out

    _, out = lax.fori_loop(0, K + 1, s2,
                           (cand_v, jnp.zeros((TM, 128), jnp.int32)))
    idx_ref[0] = out


def _knn_call(xc, xq, pT):
    return pl.pallas_call(
        _knn_body,
        grid=(B, N // TM),
        in_specs=[
            pl.BlockSpec((1, TM, 8), lambda b, m: (b, m, 0)),
            pl.BlockSpec((1, 8, N), lambda b, m: (b, 0, 0)),
            pl.BlockSpec((TM, C), lambda b, m: (b * (N // TM) + m, 0)),
        ],
        out_specs=[
            pl.BlockSpec((1, TM, 128), lambda b, m: (b, m, 0)),
            pl.BlockSpec((TM, C), lambda b, m: (b * (N // TM) + m, 0)),
        ],
        out_shape=[
            jax.ShapeDtypeStruct((B, N, 128), jnp.int32),
            jax.ShapeDtypeStruct((B * N, C), jnp.float32),
        ],
        scratch_shapes=[pltpu.VMEM((TM, _CHK, _CL), jnp.float32)],
    )(xq, xc, pT)


_NW = 32
_BPW = NUM_IDX // _NW
_CH = 128
_NCHUNK = _BPW // _CH
_PPC = _CH // K


def _sc_gather_sum(lpT, gidx):
    mesh = plsc.VectorSubcoreMesh(core_axis_name="c", subcore_axis_name="s")

    @functools.partial(
        pl.kernel,
        mesh=mesh,
        out_type=jax.ShapeDtypeStruct((B * N, C), jnp.float32),
        scratch_types=[
            pltpu.VMEM((2, _CH), jnp.int32),
            pltpu.VMEM((2, _CH, C), jnp.float32),
            pltpu.VMEM_SHARED((16, 2, _PPC, C), jnp.float32),
            pltpu.VMEM((_PPC, C), jnp.float32),
            pltpu.VMEM((_CH,), jnp.int32),
            pltpu.SemaphoreType.DMA,
            pltpu.SemaphoreType.DMA,
            pltpu.SemaphoreType.DMA,
            pltpu.SemaphoreType.DMA,
        ],
    )
    def k(lp_hbm, idx_hbm, out_hbm, idx_v, rows_v, acc_sh, zeros_v, seg_v,
          g0, g1, o0, o1):
        sid = lax.axis_index("s")
        wid = sid * 2 + lax.axis_index("c")
        ibase = wid * _BPW
        pbase = wid * (_BPW // K)
        gsem = (g0, g1)
        osem = (o0, o1)
        for r in range(_PPC):
            seg_v[pl.ds(r * K, K)] = jnp.full((K,), r, jnp.int32)
            for g in range(C // 16):
                zeros_v[r, pl.ds(g * 16, 16)] = jnp.zeros((16,), jnp.float32)
        for s in range(2):
            pltpu.sync_copy(idx_hbm.at[pl.ds(ibase + s * _CH, _CH)],
                            idx_v.at[s])
            pltpu.async_copy(lp_hbm.at[idx_v.at[s]], rows_v.at[s], gsem[s])

        @pl.loop(0, _NCHUNK, step=2)
        def _(t0):
            for s in range(2):
                t = t0 + s
                acc = acc_sh.at[sid, s]
                @pl.when(t >= 2)
                def _():
                    pltpu.make_async_copy(
                        acc, out_hbm.at[pl.ds(pbase, _PPC)],
                        osem[s]).wait()
                pltpu.sync_copy(zeros_v, acc)
                pltpu.make_async_copy(lp_hbm.at[pl.ds(0, _CH)], rows_v.at[s],
                                      gsem[s]).wait()
                pltpu.sync_copy(rows_v.at[s], acc.at[seg_v], add=True)
                pltpu.async_copy(acc,
                                 out_hbm.at[pl.ds(pbase + t * _PPC, _PPC)],
                                 osem[s])
                @pl.when(t + 2 < _NCHUNK)
                def _():
                    pltpu.sync_copy(
                        idx_hbm.at[pl.ds(ibase + (t + 2) * _CH, _CH)],
                        idx_v.at[s])
                    pltpu.async_copy(lp_hbm.at[idx_v.at[s]], rows_v.at[s],
                                     gsem[s])

        for s in range(2):
            pltpu.make_async_copy(acc_sh.at[sid, s],
                                  out_hbm.at[pl.ds(pbase, _PPC)],
                                  osem[s]).wait()

    return k(lpT, gidx)


def _mm_body(p_ref, ns_ref, w0_ref, w1_ref, w2_ref, w3_ref,
             b0_ref, b1_ref, b2_ref, b3_ref, out_ref):
    p = p_ref[...]
    lp = jnp.where(p >= 0, p, 0.01 * p)
    ns = ns_ref[...]
    t1 = (jnp.dot(lp, w0_ref[...], preferred_element_type=jnp.float32)
          + b0_ref[...]
          + jnp.dot(ns, w1_ref[...], preferred_element_type=jnp.float32)
          + K * b1_ref[...]) * (1.0 / (K + 1)) + p
    lt1 = jnp.where(t1 >= 0, t1, 0.01 * t1)
    w23 = w2_ref[...] + w3_ref[...]
    out_ref[...] = (jnp.dot(lt1, w23, preferred_element_type=jnp.float32)
                    + (b2_ref[...] + b3_ref[...])) * 0.5 + t1


def _mm_call(pT, ns, w0t, w1t, w2t, w3t, b0, b1, b2, b3):
    wspec = pl.BlockSpec((C, C), lambda i: (0, 0))
    bspec = pl.BlockSpec((1, C), lambda i: (0, 0))
    return pl.pallas_call(
        _mm_body,
        grid=(B * N // TN,),
        in_specs=[
            pl.BlockSpec((TN, C), lambda i: (i, 0)),
            pl.BlockSpec((TN, C), lambda i: (i, 0)),
            wspec, wspec, wspec, wspec,
            bspec, bspec, bspec, bspec,
        ],
        out_specs=pl.BlockSpec((TN, C), lambda i: (i, 0)),
        out_shape=jax.ShapeDtypeStruct((B * N, C), jnp.float32),
    )(pT, ns, w0t, w1t, w2t, w3t, b0, b1, b2, b3)


def kernel(xyz, points, W0, b0, W1, b1, W2, b2, W3, b3):
    xc = jnp.pad(xyz, ((0, 0), (0, 5), (0, 0)))
    xq = jnp.transpose(xc, (0, 2, 1))
    pT = jnp.transpose(points, (0, 2, 1)).reshape(B * N, C)
    idx_arr, lpT = _knn_call(xc, xq, pT)
    gidx = idx_arr[:, :, :K].reshape(NUM_IDX)
    ns = _sc_gather_sum(lpT, gidx)
    outT = _mm_call(pT, ns, W0.T, W1.T, W2.T, W3.T,
                    b0.reshape(1, C), b1.reshape(1, C),
                    b2.reshape(1, C), b3.reshape(1, C))
    return jnp.transpose(outT.reshape(B, N, C), (0, 2, 1))


# speedup vs baseline: 3.5090x; 3.5090x over previous
import functools

import jax
import jax.numpy as jnp
from jax import lax
from jax.experimental import pallas as pl
from jax.experimental.pallas import tpu as pltpu
from jax.experimental.pallas import tpu_sc as plsc

B, N, C, K = 8, 2048, 128, 16
TM = 256
TN = 512
NUM_IDX = B * N * K


TQ = 128
_NG = N // 8
_R = 10


def _knn_body(xq_ref, xc_ref, pt_ref, idx_ref, lp_ref, dscr):
    b = pl.program_id(0)
    xcT = xq_ref[0]
    acc = None
    for c in range(3):
        cc = xcT[:, c:c + 1]
        qc = xc_ref[0, c, :].reshape(1, TQ)
        d = cc - qc
        acc = d * d if acc is None else acc + d * d
    dscr[...] = acc
    m = jnp.max(dscr[...].reshape(_NG, 8, TQ), axis=0)
    pt = pt_ref[...]
    lp_ref[...] = jnp.where(pt >= 0, pt, 0.01 * pt)
    g3 = lax.broadcasted_iota(jnp.int32, (_NG, 8, 1), 0)

    vals, ixs = [], []
    for i in range(_R):
        w = dscr[...].reshape(_NG, 8, TQ)
        eq = w == m[None]
        ixg = jnp.min(jnp.where(eq, g3, _NG), axis=0)
        vals.append(m)
        ixs.append(ixg)
        if i < _R - 1:
            w2 = jnp.where(eq, -jnp.inf, w)
            dscr[...] = w2.reshape(N, TQ)
            m = jnp.max(w2, axis=0)

    wv = jnp.concatenate(vals, axis=0)
    srow = lax.broadcasted_iota(jnp.int32, (8, TQ), 0)
    nn = jnp.concatenate([ix * 8 + srow for ix in ixs], axis=0)

    out = jnp.zeros((32, TQ), jnp.int32)
    orow = lax.broadcasted_iota(jnp.int32, (32, TQ), 0)
    base = b * N
    for j in range(K + 1):
        mm = jnp.max(wv, axis=0, keepdims=True)
        eq2 = wv == mm
        sel = jnp.min(jnp.where(eq2, nn, jnp.int32(1 << 30)),
                      axis=0, keepdims=True)
        if j > 0:
            out = jnp.where(orow == j,
                            jnp.broadcast_to(sel + base, (32, TQ)), out)
        if j < K:
            wv = jnp.where(eq2 & (nn == sel), -jnp.inf, wv)
    idx_ref[0] = out


def _knn_call(xc, xq, pT):
    return pl.pallas_call(
        _knn_body,
        grid=(B, N // TQ),
        in_specs=[
            pl.BlockSpec((1, N, 8), lambda b, q: (b, 0, 0)),
            pl.BlockSpec((1, 8, TQ), lambda b, q: (b, 0, q)),
            pl.BlockSpec((TQ, C), lambda b, q: (b * (N // TQ) + q, 0)),
        ],
        out_specs=[
            pl.BlockSpec((1, 32, TQ), lambda b, q: (b, 0, q)),
            pl.BlockSpec((TQ, C), lambda b, q: (b * (N // TQ) + q, 0)),
        ],
        out_shape=[
            jax.ShapeDtypeStruct((B, 32, N), jnp.int32),
            jax.ShapeDtypeStruct((B * N, C), jnp.float32),
        ],
        scratch_shapes=[pltpu.VMEM((N, TQ), jnp.float32)],
    )(xq, xc, pT)


_NW = 32
_BPW = NUM_IDX // _NW
_CH = 128
_NCHUNK = _BPW // _CH
_PPC = _CH // K


def _sc_gather_sum(lpT, gidx):
    mesh = plsc.VectorSubcoreMesh(core_axis_name="c", subcore_axis_name="s")

    @functools.partial(
        pl.kernel,
        mesh=mesh,
        out_type=jax.ShapeDtypeStruct((B * N, C), jnp.float32),
        scratch_types=[
            pltpu.VMEM((2, _CH), jnp.int32),
            pltpu.VMEM((2, _CH, C), jnp.float32),
            pltpu.VMEM_SHARED((16, 2, _PPC, C), jnp.float32),
            pltpu.VMEM((_PPC, C), jnp.float32),
            pltpu.VMEM((_CH,), jnp.int32),
            pltpu.SemaphoreType.DMA,
            pltpu.SemaphoreType.DMA,
            pltpu.SemaphoreType.DMA,
            pltpu.SemaphoreType.DMA,
        ],
    )
    def k(lp_hbm, idx_hbm, out_hbm, idx_v, rows_v, acc_sh, zeros_v, seg_v,
          g0, g1, o0, o1):
        sid = lax.axis_index("s")
        wid = sid * 2 + lax.axis_index("c")
        ibase = wid * _BPW
        pbase = wid * (_BPW // K)
        gsem = (g0, g1)
        osem = (o0, o1)
        for r in range(_PPC):
            seg_v[pl.ds(r * K, K)] = jnp.full((K,), r, jnp.int32)
            for g in range(C // 16):
                zeros_v[r, pl.ds(g * 16, 16)] = jnp.zeros((16,), jnp.float32)
        for s in range(2):
            pltpu.sync_copy(idx_hbm.at[pl.ds(ibase + s * _CH, _CH)],
                            idx_v.at[s])
            pltpu.async_copy(lp_hbm.at[idx_v.at[s]], rows_v.at[s], gsem[s])

        @pl.loop(0, _NCHUNK, step=2)
        def _(t0):
            for s in range(2):
                t = t0 + s
                acc = acc_sh.at[sid, s]
                @pl.when(t >= 2)
                def _():
                    pltpu.make_async_copy(
                        acc, out_hbm.at[pl.ds(pbase, _PPC)],
                        osem[s]).wait()
                pltpu.sync_copy(zeros_v, acc)
                pltpu.make_async_copy(lp_hbm.at[pl.ds(0, _CH)], rows_v.at[s],
                                      gsem[s]).wait()
                pltpu.sync_copy(rows_v.at[s], acc.at[seg_v], add=True)
                pltpu.async_copy(acc,
                                 out_hbm.at[pl.ds(pbase + t * _PPC, _PPC)],
                                 osem[s])
                @pl.when(t + 2 < _NCHUNK)
                def _():
                    pltpu.sync_copy(
                        idx_hbm.at[pl.ds(ibase + (t + 2) * _CH, _CH)],
                        idx_v.at[s])
                    pltpu.async_copy(lp_hbm.at[idx_v.at[s]], rows_v.at[s],
                                     gsem[s])

        for s in range(2):
            pltpu.make_async_copy(acc_sh.at[sid, s],
                                  out_hbm.at[pl.ds(pbase, _PPC)],
                                  osem[s]).wait()

    return k(lpT, gidx)


def _mm_body(p_ref, ns_ref, w0_ref, w1_ref, w2_ref, w3_ref,
             b0_ref, b1_ref, b2_ref, b3_ref, out_ref):
    p = p_ref[...]
    lp = jnp.where(p >= 0, p, 0.01 * p)
    ns = ns_ref[...]
    t1 = (jnp.dot(lp, w0_ref[...], preferred_element_type=jnp.float32)
          + b0_ref[...]
          + jnp.dot(ns, w1_ref[...], preferred_element_type=jnp.float32)
          + K * b1_ref[...]) * (1.0 / (K + 1)) + p
    lt1 = jnp.where(t1 >= 0, t1, 0.01 * t1)
    w23 = w2_ref[...] + w3_ref[...]
    out_ref[...] = (jnp.dot(lt1, w23, preferred_element_type=jnp.float32)
                    + (b2_ref[...] + b3_ref[...])) * 0.5 + t1


def _mm_call(pT, ns, w0t, w1t, w2t, w3t, b0, b1, b2, b3):
    wspec = pl.BlockSpec((C, C), lambda i: (0, 0))
    bspec = pl.BlockSpec((1, C), lambda i: (0, 0))
    return pl.pallas_call(
        _mm_body,
        grid=(B * N // TN,),
        in_specs=[
            pl.BlockSpec((TN, C), lambda i: (i, 0)),
            pl.BlockSpec((TN, C), lambda i: (i, 0)),
            wspec, wspec, wspec, wspec,
            bspec, bspec, bspec, bspec,
        ],
        out_specs=pl.BlockSpec((TN, C), lambda i: (i, 0)),
        out_shape=jax.ShapeDtypeStruct((B * N, C), jnp.float32),
    )(pT, ns, w0t, w1t, w2t, w3t, b0, b1, b2, b3)


def kernel(xyz, points, W0, b0, W1, b1, W2, b2, W3, b3):
    xc = jnp.pad(xyz, ((0, 0), (0, 5), (0, 0)))
    xq = jnp.transpose(xc, (0, 2, 1))
    pT = jnp.transpose(points, (0, 2, 1)).reshape(B * N, C)
    idx_arr, lpT = _knn_call(xc, xq, pT)
    gidx = jnp.transpose(idx_arr[:, 1:K + 1, :], (0, 2, 1)).reshape(NUM_IDX)
    ns = _sc_gather_sum(lpT, gidx)
    outT = _mm_call(pT, ns, W0.T, W1.T, W2.T, W3.T,
                    b0.reshape(1, C), b1.reshape(1, C),
                    b2.reshape(1, C), b3.reshape(1, C))
    return jnp.transpose(outT.reshape(B, N, C), (0, 2, 1))
